# baseline (speedup 1.0000x reference)
import functools

import jax
import jax.numpy as jnp
from jax import lax
from jax.experimental import pallas as pl
from jax.experimental.pallas import tpu as pltpu
from jax.experimental.pallas import tpu_sc as plsc

N_TOKEN = 1000000
D_EMBED = 64
BATCH = 4096
HIST = 200
TOT = BATCH * HIST

NC = 2
NS = 16
NW = NC * NS
IW = 128
G = 4
BLOCK = G * IW
ROWS128_PER_W = TOT // (NW * IW)
BLOCKS_PER_W = ROWS128_PER_W // G

_mesh = plsc.VectorSubcoreMesh(core_axis_name="c", subcore_axis_name="s")


@functools.partial(
    pl.kernel,
    mesh=_mesh,
    compiler_params=pltpu.CompilerParams(use_tc_tiling_on_sc=False),
    out_type=jax.ShapeDtypeStruct((TOT, D_EMBED), jnp.float32),
    scratch_types=[
        pltpu.VMEM((G, IW), jnp.int32),
        pltpu.VMEM((BLOCK, D_EMBED), jnp.float32),
        pltpu.SemaphoreType.DMA,
    ],
)
def _emb_lookup(idx_hbm, table_hbm, out_hbm, idx_v, rows_v, sem):
    wid = lax.axis_index("s") * NC + lax.axis_index("c")
    base128 = wid * ROWS128_PER_W

    def block_fn(b, carry):
        start128 = base128 + b * G
        pltpu.sync_copy(idx_hbm.at[pl.ds(start128, G)], idx_v)
        descs = []
        for j in range(G):
            descs.append(
                pltpu.async_copy(
                    table_hbm.at[idx_v.at[j]],
                    rows_v.at[pl.ds(j * IW, IW)],
                    sem,
                )
            )
        for d in descs:
            d.wait()
        pltpu.sync_copy(rows_v, out_hbm.at[pl.ds(start128 * IW, BLOCK)])
        return carry

    lax.fori_loop(0, BLOCKS_PER_W, block_fn, 0)


def kernel(inputs, lookup_table):
    idx2d = inputs.reshape(TOT // IW, IW)
    flat = _emb_lookup(idx2d, lookup_table)
    return flat.reshape(BATCH, HIST, D_EMBED), lookup_table


# speedup vs baseline: 1.0377x; 1.0377x over previous
import functools

import jax
import jax.numpy as jnp
from jax import lax
from jax.experimental import pallas as pl
from jax.experimental.pallas import tpu as pltpu
from jax.experimental.pallas import tpu_sc as plsc

N_TOKEN = 1000000
D_EMBED = 64
BATCH = 4096
HIST = 200
TOT = BATCH * HIST

NC = 2
NS = 16
NW = NC * NS
IW = 128
G = 4
BLOCK = G * IW
ROWS128_PER_W = TOT // (NW * IW)
NB = ROWS128_PER_W // G

_mesh = plsc.VectorSubcoreMesh(core_axis_name="c", subcore_axis_name="s")


@functools.partial(
    pl.kernel,
    mesh=_mesh,
    compiler_params=pltpu.CompilerParams(use_tc_tiling_on_sc=False),
    out_type=jax.ShapeDtypeStruct((TOT, D_EMBED), jnp.float32),
    scratch_types=[
        pltpu.VMEM((ROWS128_PER_W, IW), jnp.int32),
        pltpu.VMEM((2, BLOCK, D_EMBED), jnp.float32),
        pltpu.SemaphoreType.DMA((2,)),
        pltpu.SemaphoreType.DMA((2,)),
    ],
)
def _emb_lookup(idx_hbm, table_hbm, out_hbm, idx_v, rows_v, gsem, osem):
    wid = lax.axis_index("s") * NC + lax.axis_index("c")
    base128 = wid * ROWS128_PER_W
    pltpu.sync_copy(idx_hbm.at[pl.ds(base128, ROWS128_PER_W)], idx_v)

    def fire_gather(b, slot):
        for j in range(G):
            pltpu.async_copy(
                table_hbm.at[idx_v.at[b * G + j]],
                rows_v.at[slot, pl.ds(j * IW, IW)],
                gsem.at[slot],
            )

    def wait_gather(slot):
        pltpu.make_async_copy(
            out_hbm.at[pl.ds(0, BLOCK)], rows_v.at[slot], gsem.at[slot]
        ).wait()

    def fire_wb(b, slot):
        pltpu.async_copy(
            rows_v.at[slot],
            out_hbm.at[pl.ds((base128 + b * G) * IW, BLOCK)],
            osem.at[slot],
        )

    def wait_wb(slot):
        pltpu.make_async_copy(
            rows_v.at[slot], out_hbm.at[pl.ds(0, BLOCK)], osem.at[slot]
        ).wait()

    fire_gather(0, 0)
    fire_gather(1, 1)

    def body(g, carry):
        for s in range(2):
            wait_gather(s)
            fire_wb(2 * g + s, s)
        for s in range(2):
            nxt = 2 * g + 2 + s

            @pl.when(nxt < NB)
            def _():
                wait_wb(s)
                fire_gather(nxt, s)

        return carry

    lax.fori_loop(0, NB // 2, body, 0)
    wait_wb(0)
    wait_wb(1)


def kernel(inputs, lookup_table):
    idx2d = inputs.reshape(TOT // IW, IW)
    flat = _emb_lookup(idx2d, lookup_table)
    return flat.reshape(BATCH, HIST, D_EMBED), lookup_table
